# baseline (speedup 1.0000x reference)
import functools

import jax
import jax.numpy as jnp
from jax import lax
from jax.experimental import pallas as pl
from jax.experimental.pallas import tpu as pltpu
from jax.experimental.pallas import tpu_sc as plsc

B = 4096
F = 26
VOCAB = 100000
D = 32

NC = 2
NS = 16
NW = NC * NS
L = 16

R = B * F
RPW = R // NW
CH = 128
NCH = RPW // CH

_mesh = plsc.VectorSubcoreMesh(core_axis_name="c", subcore_axis_name="s")


@functools.partial(
    pl.kernel,
    mesh=_mesh,
    out_type=jax.ShapeDtypeStruct((NW, NCH, CH, D), jnp.float32),
    scratch_types=[
        pltpu.VMEM((NCH, CH), jnp.int32),
        pltpu.VMEM((NCH, CH), jnp.int32),
        pltpu.VMEM((NCH, CH, D), jnp.float32),
        pltpu.SemaphoreType.DMA,
    ],
    compiler_params=pltpu.CompilerParams(use_tc_tiling_on_sc=False),
)
def _emb(x_hbm, off_hbm, tab_hbm, out_hbm, idx_v, off_v, rows_v, sem):
    wid = lax.axis_index("s") * NC + lax.axis_index("c")
    pltpu.sync_copy(x_hbm.at[wid], idx_v)
    pltpu.sync_copy(off_hbm, off_v)
    for j in range(NCH):
        for c in range(CH // L):
            s = pl.ds(c * L, L)
            idx_v[j, s] = idx_v[j, s] + off_v[j, s]
    copies = [
        pltpu.async_copy(tab_hbm.at[idx_v.at[j]], rows_v.at[j], sem)
        for j in range(NCH)
    ]
    for cp in copies:
        cp.wait()
    pltpu.sync_copy(rows_v, out_hbm.at[wid])


def kernel(X, tables):
    xw = X.reshape(NW, NCH, CH)
    tab = tables.reshape(F * VOCAB, D)
    offs = ((jnp.arange(RPW, dtype=jnp.int32) % F) * VOCAB).reshape(NCH, CH)
    out = _emb(xw, offs, tab)
    return out.reshape(B, F, D)


# speedup vs baseline: 1.8442x; 1.8442x over previous
import functools

import jax
import jax.numpy as jnp
from jax import lax
from jax.experimental import pallas as pl
from jax.experimental.pallas import tpu as pltpu
from jax.experimental.pallas import tpu_sc as plsc

B = 4096
F = 26
VOCAB = 100000
D = 32

NC = 2
NS = 16
NW = NC * NS
L = 16

R = F * D
RPW = R // NW
CH = 128
NCH = B // CH

_mesh = plsc.VectorSubcoreMesh(core_axis_name="c", subcore_axis_name="s")


@functools.partial(
    pl.kernel,
    mesh=_mesh,
    out_type=jax.ShapeDtypeStruct((R, B), jnp.float32),
    scratch_types=[
        pltpu.VMEM((B,), jnp.int32),
        pltpu.VMEM((B,), jnp.int32),
        pltpu.VMEM((B,), jnp.float32),
        pltpu.SemaphoreType.DMA,
        pltpu.SemaphoreType.DMA,
    ],
)
def _emb(xt_hbm, tab_hbm, out_hbm, xrow_v, idx_v, row_v, sem_x, sem_g):
    wid = lax.axis_index("s") * NC + lax.axis_index("c")
    r0 = wid * RPW

    @pl.loop(0, RPW)
    def _row(i):
        r = r0 + i
        f = r // D
        pltpu.async_copy(xt_hbm.at[f], xrow_v, sem_x).wait()
        base = r * VOCAB

        @pl.loop(0, B // L)
        def _add(c):
            s = pl.ds(c * L, L)
            idx_v[s] = xrow_v[s] + base

        copies = [
            pltpu.async_copy(
                tab_hbm.at[idx_v.at[pl.ds(j * CH, CH)]],
                row_v.at[pl.ds(j * CH, CH)],
                sem_g,
            )
            for j in range(NCH)
        ]
        for cp in copies:
            cp.wait()
        pltpu.sync_copy(row_v, out_hbm.at[r])


def kernel(X, tables):
    xt = X.T
    tab = tables.transpose(0, 2, 1).reshape(R * VOCAB)
    out = _emb(xt, tab)
    return out.reshape(F, D, B).transpose(2, 0, 1)


# speedup vs baseline: 4.9014x; 2.6578x over previous
import functools

import jax
import jax.numpy as jnp
from jax import lax
from jax.experimental import pallas as pl
from jax.experimental.pallas import tpu as pltpu
from jax.experimental.pallas import tpu_sc as plsc

B = 4096
F = 26
VOCAB = 100000
D = 32

NC = 2
NS = 16
L = 16
G = D // 8
FPC = F // NC
NGRP = FPC * G
BPT = B // NS
CH = 128
VMAIN = 99968
VOCABP = VMAIN + 128

_mesh = plsc.VectorSubcoreMesh(core_axis_name="c", subcore_axis_name="s")


@functools.partial(
    pl.kernel,
    mesh=_mesh,
    out_type=jax.ShapeDtypeStruct((F * D, B), jnp.float32),
    scratch_types=[
        pltpu.VMEM_SHARED((8 * VOCABP,), jnp.float32),
        pltpu.VMEM_SHARED((8 * VOCABP,), jnp.float32),
        pltpu.VMEM((8, BPT), jnp.int32),
        pltpu.VMEM((8, BPT), jnp.int32),
        pltpu.VMEM((8, BPT), jnp.float32),
        pltpu.SemaphoreType.DMA,
        pltpu.SemaphoreType.DMA,
    ],
    compiler_params=pltpu.CompilerParams(use_tc_tiling_on_sc=True),
)
def _emb(xt_hbm, tab_hbm, tail_hbm, out_hbm, sp0, sp1, xblk_v, idx_v, plane_v,
         sem_s, sem_g):
    cid = lax.axis_index("c")
    sid = lax.axis_index("s")
    fbase = cid * FPC
    bcol = pl.multiple_of(sid * BPT, 128)

    def stage(fq, gq, sp):
        for sl in range(8):
            pltpu.async_copy(
                tab_hbm.at[fq, gq, sl, pl.ds(0, VMAIN)],
                sp.at[pl.ds(sl * VOCABP, VMAIN)],
                sem_s,
            )
            pltpu.async_copy(
                tail_hbm.at[fq, gq, sl],
                sp.at[pl.ds(sl * VOCABP + VMAIN, 128)],
                sem_s,
            )

    def stage_wait(sp):
        for sl in range(8):
            pltpu.make_async_copy(
                tab_hbm.at[0, 0, sl, pl.ds(0, VMAIN)],
                sp.at[pl.ds(sl * VOCABP, VMAIN)],
                sem_s,
            ).wait()
            pltpu.make_async_copy(
                tail_hbm.at[0, 0, sl],
                sp.at[pl.ds(sl * VOCABP + VMAIN, 128)],
                sem_s,
            ).wait()

    @pl.when(sid == 0)
    def _():
        stage(fbase, 0, sp0)

    @pl.loop(0, NGRP, step=2)
    def _grp(gi):
        for h, sp in ((0, sp0), (1, sp1)):
            gq = gi + h
            f = fbase + gq // G
            g = gq % G

            @pl.when(sid == 0)
            def _():
                stage_wait(sp)

            plsc.subcore_barrier()

            @pl.when(jnp.logical_and(sid == 0, gq + 1 < NGRP))
            def _():
                nxt = gq + 1
                stage(fbase + nxt // G, nxt % G, sp1 if h == 0 else sp0)

            @pl.when(g == 0)
            def _():
                frow8 = pl.multiple_of((f // 8) * 8, 8)
                pltpu.sync_copy(
                    xt_hbm.at[pl.ds(frow8, 8), pl.ds(bcol, BPT)], xblk_v
                )

            fr = f % 8

            @pl.loop(0, BPT // L)
            def _add(c):
                s = pl.ds(c * L, L)
                v16 = xblk_v[fr, s]
                for sl in range(8):
                    idx_v[sl, s] = v16 + (sl * VOCABP)

            copies = [
                pltpu.async_copy(
                    sp.at[idx_v.at[sl, pl.ds(j * CH, CH)]],
                    plane_v.at[sl, pl.ds(j * CH, CH)],
                    sem_g,
                )
                for sl in range(8)
                for j in range(BPT // CH)
            ]
            for cp in copies:
                cp.wait()

            row0 = pl.multiple_of(f * D + g * 8, 8)
            pltpu.sync_copy(
                plane_v, out_hbm.at[pl.ds(row0, 8), pl.ds(bcol, BPT)]
            )

            plsc.subcore_barrier()


def kernel(X, tables):
    xt = jnp.pad(X.T, ((0, 32 - F), (0, 0)))
    tab = tables.transpose(0, 2, 1).reshape(F, G, 8, VOCAB)
    tail = jnp.pad(
        tables[:, VMAIN:, :].transpose(0, 2, 1), ((0, 0), (0, 0), (0, 96))
    ).reshape(F, G, 8, 128)
    out = _emb(xt, tab, tail)
    return out.reshape(F, D, B).transpose(2, 0, 1)


# speedup vs baseline: 4.9040x; 1.0005x over previous
import functools

import jax
import jax.numpy as jnp
from jax import lax
from jax.experimental import pallas as pl
from jax.experimental.pallas import tpu as pltpu
from jax.experimental.pallas import tpu_sc as plsc

B = 4096
F = 26
VOCAB = 100000
D = 32

NC = 2
NS = 16
L = 16
G = D // 8
FPC = F // NC
NGRP = FPC * G
BPT = B // NS
CH = 128
VMAIN = 99968
VOCABP = VMAIN + 128

_mesh = plsc.VectorSubcoreMesh(core_axis_name="c", subcore_axis_name="s")


@functools.partial(
    pl.kernel,
    mesh=_mesh,
    out_type=jax.ShapeDtypeStruct((F * D, B), jnp.float32),
    scratch_types=[
        pltpu.VMEM_SHARED((8 * VOCABP,), jnp.float32),
        pltpu.VMEM_SHARED((8 * VOCABP,), jnp.float32),
        pltpu.VMEM((8, BPT), jnp.int32),
        pltpu.VMEM((8, BPT), jnp.int32),
        pltpu.VMEM((8, BPT), jnp.float32),
        pltpu.SemaphoreType.DMA,
        pltpu.SemaphoreType.DMA,
    ],
    compiler_params=pltpu.CompilerParams(use_tc_tiling_on_sc=True),
)
def _emb(xt_hbm, tab_hbm, tail_hbm, out_hbm, sp0, sp1, xblk_v, idx_v, plane_v,
         sem_s, sem_g):
    cid = lax.axis_index("c")
    sid = lax.axis_index("s")
    fbase = cid * FPC
    bcol = pl.multiple_of(sid * BPT, 128)

    def stage(fq, gq, sp):
        for sl in range(8):
            pltpu.async_copy(
                tab_hbm.at[fq, gq, sl, pl.ds(0, VMAIN)],
                sp.at[pl.ds(sl * VOCABP, VMAIN)],
                sem_s,
            )
            pltpu.async_copy(
                tail_hbm.at[fq, gq, sl],
                sp.at[pl.ds(sl * VOCABP + VMAIN, 128)],
                sem_s,
            )

    def stage_wait(sp):
        for sl in range(8):
            pltpu.make_async_copy(
                tab_hbm.at[0, 0, sl, pl.ds(0, VMAIN)],
                sp.at[pl.ds(sl * VOCABP, VMAIN)],
                sem_s,
            ).wait()
            pltpu.make_async_copy(
                tail_hbm.at[0, 0, sl],
                sp.at[pl.ds(sl * VOCABP + VMAIN, 128)],
                sem_s,
            ).wait()

    @pl.when(sid == 0)
    def _():
        stage(fbase, 0, sp0)

    @pl.loop(0, NGRP, step=2)
    def _grp(gi):
        for h, sp in ((0, sp0), (1, sp1)):
            gq = gi + h
            f = fbase + gq // G
            g = gq % G

            @pl.when(sid == 0)
            def _():
                stage_wait(sp)

            plsc.subcore_barrier()

            @pl.when(jnp.logical_and(sid == 0, gq + 1 < NGRP))
            def _():
                nxt = gq + 1
                stage(fbase + nxt // G, nxt % G, sp1 if h == 0 else sp0)

            @pl.when(g == 0)
            def _():
                frow8 = pl.multiple_of((f // 8) * 8, 8)
                pltpu.sync_copy(
                    xt_hbm.at[pl.ds(frow8, 8), pl.ds(bcol, BPT)], xblk_v
                )

            fr = f % 8

            @pl.loop(0, BPT // L)
            def _add(c):
                s = pl.ds(c * L, L)
                v16 = xblk_v[fr, s]
                for sl in range(8):
                    idx_v[sl, s] = v16 + (sl * VOCABP)

            copies = [
                pltpu.async_copy(
                    sp.at[idx_v.at[sl, pl.ds(j * CH, CH)]],
                    plane_v.at[sl, pl.ds(j * CH, CH)],
                    sem_g,
                )
                for sl in range(8)
                for j in range(BPT // CH)
            ]
            for cp in copies:
                cp.wait()

            row0 = pl.multiple_of(f * D + g * 8, 8)
            pltpu.sync_copy(
                plane_v, out_hbm.at[pl.ds(row0, 8), pl.ds(bcol, BPT)]
            )


def kernel(X, tables):
    xt = jnp.pad(X.T, ((0, 32 - F), (0, 0)))
    tab = tables.transpose(0, 2, 1).reshape(F, G, 8, VOCAB)
    tail = jnp.pad(
        tables[:, VMAIN:, :].transpose(0, 2, 1), ((0, 0), (0, 0), (0, 96))
    ).reshape(F, G, 8, 128)
    out = _emb(xt, tab, tail)
    return out.reshape(F, D, B).transpose(2, 0, 1)
